# Initial kernel scaffold; baseline (speedup 1.0000x reference)
import jax
import jax.numpy as jnp
from jax.experimental import pallas as pl


def kernel(x, adj, W1, b1, Wmu, bmu, Wsig, bsig):
    raise NotImplementedError("write your pallas kernel here")


# speedup vs baseline: 10.7141x; 10.7141x over previous
import functools

import jax
import jax.numpy as jnp
from jax import lax
from jax.experimental import pallas as pl
from jax.experimental.pallas import tpu as pltpu
from jax.experimental.pallas import tpu_sc as plsc

_N = 10000
_E = 320000
_F = 128
_Z = 64
_N_PAD = 10240
_E_PAD = 327680
_NW = 32
_CHUNKS = 80
_ROWS_PER_TILE = _N_PAD // 16
_DUMMY = 10200


_sc_mesh = plsc.VectorSubcoreMesh(core_axis_name="c", subcore_axis_name="s")


@functools.partial(
    pl.kernel,
    out_type=jax.ShapeDtypeStruct((2, _N_PAD, _F), jnp.float32),
    mesh=_sc_mesh,
    scratch_types=[
        pltpu.VMEM((_CHUNKS, 128), jnp.int32),
        pltpu.VMEM((128, _F), jnp.float32),
        pltpu.VMEM_SHARED((_N_PAD, _F), jnp.float32),
    ],
)
def _deg_kernel(dst_hbm, zeros_hbm, out, dst_v, ones_v, acc_sh):
    c = lax.axis_index("c")
    s = lax.axis_index("s")
    w = s * 2 + c
    row0 = s * _ROWS_PER_TILE
    rows = pl.ds(row0, _ROWS_PER_TILE)
    pltpu.sync_copy(zeros_hbm.at[rows], acc_sh.at[rows])

    def ofill(i, carry):
        for m in range(8):
            ones_v[i, pl.ds(m * 16, 16)] = jnp.ones((16,), jnp.float32)
        return carry

    lax.fori_loop(0, 128, ofill, 0)
    pltpu.sync_copy(dst_hbm.at[pl.ds(w * _CHUNKS, _CHUNKS)], dst_v)
    plsc.subcore_barrier()

    def body(j, carry):
        pltpu.sync_copy(ones_v, acc_sh.at[dst_v.at[j]], add=True)
        return carry

    lax.fori_loop(0, _CHUNKS, body, 0)
    plsc.subcore_barrier()
    pltpu.sync_copy(acc_sh.at[rows], out.at[c, rows])


@functools.partial(
    pl.kernel,
    out_type=jax.ShapeDtypeStruct((2, _N_PAD, _F), jnp.float32),
    mesh=_sc_mesh,
    scratch_types=[
        pltpu.VMEM((_CHUNKS, 128), jnp.int32),
        pltpu.VMEM((_CHUNKS, 128), jnp.int32),
        pltpu.VMEM((128, _F), jnp.float32),
        pltpu.VMEM_SHARED((_N_PAD, _F), jnp.float32),
        pltpu.SemaphoreType.DMA,
    ],
)
def _agg_kernel(src_hbm, dst_hbm, feat_hbm, zeros_hbm, out,
                src_v, dst_v, rows_v, acc_sh, sem):
    c = lax.axis_index("c")
    s = lax.axis_index("s")
    w = s * 2 + c
    row0 = s * _ROWS_PER_TILE
    rows = pl.ds(row0, _ROWS_PER_TILE)
    pltpu.sync_copy(zeros_hbm.at[rows], acc_sh.at[rows])
    pltpu.sync_copy(src_hbm.at[pl.ds(w * _CHUNKS, _CHUNKS)], src_v)
    pltpu.sync_copy(dst_hbm.at[pl.ds(w * _CHUNKS, _CHUNKS)], dst_v)
    plsc.subcore_barrier()

    def body(j, carry):
        pltpu.async_copy(feat_hbm.at[src_v.at[j]], rows_v, sem).wait()
        pltpu.sync_copy(rows_v, acc_sh.at[dst_v.at[j]], add=True)
        return carry

    lax.fori_loop(0, _CHUNKS, body, 0)
    plsc.subcore_barrier()
    pltpu.sync_copy(acc_sh.at[rows], out.at[c, rows])


_BM = 1024
_GRID = (_N_PAD // _BM,)


def _scale_body(x_ref, da_ref, db_ref, xs_ref, dv_ref):
    deg = da_ref[0] + db_ref[0] + 1.0
    dinv = lax.rsqrt(deg)
    xs_ref[...] = x_ref[...] * dinv
    dv_ref[...] = dinv


def _hidden_body(pa_ref, pb_ref, xs_ref, dv_ref, w1_ref, b1_ref, hs_ref):
    dv = dv_ref[...]
    t = (pa_ref[0] + pb_ref[0] + xs_ref[...]) * dv
    h = jnp.dot(t, w1_ref[...], preferred_element_type=jnp.float32) + b1_ref[...]
    hidden = jnp.maximum(h, 0.0) + jnp.log1p(jnp.exp(-jnp.abs(h)))
    hs_ref[...] = hidden * dv


def _out_body(qa_ref, qb_ref, hs_ref, dv_ref, wmu_ref, bmu_ref,
              wsig_ref, bsig_ref, zl_ref, zs_ref):
    t = (qa_ref[0] + qb_ref[0] + hs_ref[...]) * dv_ref[...]
    zl_ref[...] = jnp.dot(t, wmu_ref[...],
                          preferred_element_type=jnp.float32) + bmu_ref[...]
    zs_ref[...] = jnp.exp(jnp.dot(t, wsig_ref[...],
                                  preferred_element_type=jnp.float32) + bsig_ref[...])


def _row_spec(width):
    return pl.BlockSpec((_BM, width), lambda i: (i, 0))


def _part_spec(cidx, width):
    return pl.BlockSpec((1, _BM, width), lambda i, c=cidx: (c, i, 0))


def _full_spec(r, ccols):
    return pl.BlockSpec((r, ccols), lambda i: (0, 0))


_scale_call = pl.pallas_call(
    _scale_body,
    grid=_GRID,
    in_specs=[_row_spec(_F), _part_spec(0, _F), _part_spec(1, _F)],
    out_specs=[_row_spec(_F), _row_spec(_F)],
    out_shape=[jax.ShapeDtypeStruct((_N_PAD, _F), jnp.float32)] * 2,
)

_hidden_call = pl.pallas_call(
    _hidden_body,
    grid=_GRID,
    in_specs=[_part_spec(0, _F), _part_spec(1, _F), _row_spec(_F), _row_spec(_F),
              _full_spec(_F, _F), _full_spec(1, _F)],
    out_specs=[_row_spec(_F)],
    out_shape=[jax.ShapeDtypeStruct((_N_PAD, _F), jnp.float32)],
)

_out_call = pl.pallas_call(
    _out_body,
    grid=_GRID,
    in_specs=[_part_spec(0, _F), _part_spec(1, _F), _row_spec(_F), _row_spec(_F),
              _full_spec(_F, _Z), _full_spec(1, _Z), _full_spec(_F, _Z), _full_spec(1, _Z)],
    out_specs=[_row_spec(_Z), _row_spec(_Z)],
    out_shape=[jax.ShapeDtypeStruct((_N, _Z), jnp.float32)] * 2,
)


def kernel(x, adj, W1, b1, Wmu, bmu, Wsig, bsig):
    src = adj[0].astype(jnp.int32)
    dst = adj[1].astype(jnp.int32)
    fill = jnp.full((_E_PAD - _E,), _DUMMY, dtype=jnp.int32)
    src2 = jnp.concatenate([src, fill]).reshape(_NW * _CHUNKS, 128)
    dst2 = jnp.concatenate([dst, fill]).reshape(_NW * _CHUNKS, 128)
    x_p = jnp.pad(x, ((0, _N_PAD - _N), (0, 0)))
    zeros_f = jnp.zeros((_N_PAD, _F), jnp.float32)

    deg2 = _deg_kernel(dst2, zeros_f)
    xs, dv = _scale_call(x_p, deg2, deg2)
    p = _agg_kernel(src2, dst2, xs, zeros_f)
    (hs,) = _hidden_call(p, p, xs, dv, W1, b1.reshape(1, _F))
    q = _agg_kernel(src2, dst2, hs, zeros_f)
    z_loc, z_scale = _out_call(q, q, hs, dv, Wmu, bmu.reshape(1, _Z),
                               Wsig, bsig.reshape(1, _Z))
    return z_loc, z_scale


# speedup vs baseline: 12.0289x; 1.1227x over previous
import functools

import jax
import jax.numpy as jnp
from jax import lax
from jax.experimental import pallas as pl
from jax.experimental.pallas import tpu as pltpu
from jax.experimental.pallas import tpu_sc as plsc

_N = 10000
_E = 320000
_F = 128
_Z = 64
_N_PAD = 10240
_E_PAD = 327680
_NW = 32
_CHUNKS = 80
_ROWS_PER_TILE = _N_PAD // 16
_DUMMY = 10200


_sc_mesh = plsc.VectorSubcoreMesh(core_axis_name="c", subcore_axis_name="s")


@functools.partial(
    pl.kernel,
    out_type=jax.ShapeDtypeStruct((2, _N_PAD, _F), jnp.float32),
    mesh=_sc_mesh,
    scratch_types=[
        pltpu.VMEM((_CHUNKS, 128), jnp.int32),
        pltpu.VMEM((128, _F), jnp.float32),
        pltpu.VMEM_SHARED((_N_PAD, _F), jnp.float32),
    ],
)
def _deg_kernel(dst_hbm, zeros_hbm, out, dst_v, ones_v, acc_sh):
    c = lax.axis_index("c")
    s = lax.axis_index("s")
    w = s * 2 + c
    row0 = s * _ROWS_PER_TILE
    rows = pl.ds(row0, _ROWS_PER_TILE)
    pltpu.sync_copy(zeros_hbm.at[rows], acc_sh.at[rows])

    def ofill(i, carry):
        for m in range(8):
            ones_v[i, pl.ds(m * 16, 16)] = jnp.ones((16,), jnp.float32)
        return carry

    lax.fori_loop(0, 128, ofill, 0)
    pltpu.sync_copy(dst_hbm.at[pl.ds(w * _CHUNKS, _CHUNKS)], dst_v)
    plsc.subcore_barrier()

    def body(j, carry):
        pltpu.sync_copy(ones_v, acc_sh.at[dst_v.at[j]], add=True)
        return carry

    lax.fori_loop(0, _CHUNKS, body, 0)
    plsc.subcore_barrier()
    pltpu.sync_copy(acc_sh.at[rows], out.at[c, rows])


_NBUF = 2
_HALF = _CHUNKS // 2


@functools.partial(
    pl.kernel,
    out_type=jax.ShapeDtypeStruct((2, _N_PAD, _F), jnp.float32),
    mesh=_sc_mesh,
    scratch_types=[
        pltpu.VMEM((_HALF, 128), jnp.int32),
        pltpu.VMEM((_HALF, 128), jnp.int32),
    ]
    + [pltpu.VMEM((128, _F), jnp.float32) for _ in range(_NBUF)]
    + [pltpu.VMEM_SHARED((_N_PAD, _F), jnp.float32)]
    + [pltpu.SemaphoreType.DMA for _ in range(_NBUF)],
)
def _agg_kernel(src_hbm, dst_hbm, feat_hbm, zeros_hbm, out,
                src_v, dst_v, b0, b1, acc_sh, s0, s1):
    c = lax.axis_index("c")
    s = lax.axis_index("s")
    w = s * 2 + c
    row0 = s * _ROWS_PER_TILE
    rows = pl.ds(row0, _ROWS_PER_TILE)
    bufs = (b0, b1)
    sems = (s0, s1)
    pltpu.sync_copy(zeros_hbm.at[rows], acc_sh.at[rows])
    plsc.subcore_barrier()

    for h in range(2):
        hbase = w * _CHUNKS + h * _HALF
        pltpu.sync_copy(src_hbm.at[pl.ds(hbase, _HALF)], src_v)
        pltpu.sync_copy(dst_hbm.at[pl.ds(hbase, _HALF)], dst_v)
        for b in range(_NBUF):
            pltpu.async_copy(feat_hbm.at[src_v.at[b]], bufs[b], sems[b])

        def body(k, carry):
            for b in range(_NBUF):
                j = k * _NBUF + b
                pltpu.make_async_copy(feat_hbm.at[src_v.at[j]], bufs[b],
                                      sems[b]).wait()
                pltpu.sync_copy(bufs[b], acc_sh.at[dst_v.at[j]], add=True)
                jn = j + _NBUF

                @pl.when(jn < _HALF)
                def _():
                    pltpu.async_copy(feat_hbm.at[src_v.at[jn]], bufs[b],
                                     sems[b])

            return carry

        lax.fori_loop(0, _HALF // _NBUF, body, 0)

    plsc.subcore_barrier()
    pltpu.sync_copy(acc_sh.at[rows], out.at[c, rows])


_BM = 1024
_GRID = (_N_PAD // _BM,)


def _scale_body(x_ref, da_ref, db_ref, xs_ref, dv_ref):
    deg = da_ref[0] + db_ref[0] + 1.0
    dinv = lax.rsqrt(deg)
    xs_ref[...] = x_ref[...] * dinv
    dv_ref[...] = dinv


def _hidden_body(pa_ref, pb_ref, xs_ref, dv_ref, w1_ref, b1_ref, hs_ref):
    dv = dv_ref[...]
    t = (pa_ref[0] + pb_ref[0] + xs_ref[...]) * dv
    h = jnp.dot(t, w1_ref[...], preferred_element_type=jnp.float32) + b1_ref[...]
    hidden = jnp.maximum(h, 0.0) + jnp.log1p(jnp.exp(-jnp.abs(h)))
    hs_ref[...] = hidden * dv


def _out_body(qa_ref, qb_ref, hs_ref, dv_ref, wmu_ref, bmu_ref,
              wsig_ref, bsig_ref, zl_ref, zs_ref):
    t = (qa_ref[0] + qb_ref[0] + hs_ref[...]) * dv_ref[...]
    zl_ref[...] = jnp.dot(t, wmu_ref[...],
                          preferred_element_type=jnp.float32) + bmu_ref[...]
    zs_ref[...] = jnp.exp(jnp.dot(t, wsig_ref[...],
                                  preferred_element_type=jnp.float32) + bsig_ref[...])


def _row_spec(width):
    return pl.BlockSpec((_BM, width), lambda i: (i, 0))


def _part_spec(cidx, width):
    return pl.BlockSpec((1, _BM, width), lambda i, c=cidx: (c, i, 0))


def _full_spec(r, ccols):
    return pl.BlockSpec((r, ccols), lambda i: (0, 0))


_scale_call = pl.pallas_call(
    _scale_body,
    grid=_GRID,
    in_specs=[_row_spec(_F), _part_spec(0, _F), _part_spec(1, _F)],
    out_specs=[_row_spec(_F), _row_spec(_F)],
    out_shape=[jax.ShapeDtypeStruct((_N_PAD, _F), jnp.float32)] * 2,
)

_hidden_call = pl.pallas_call(
    _hidden_body,
    grid=_GRID,
    in_specs=[_part_spec(0, _F), _part_spec(1, _F), _row_spec(_F), _row_spec(_F),
              _full_spec(_F, _F), _full_spec(1, _F)],
    out_specs=[_row_spec(_F)],
    out_shape=[jax.ShapeDtypeStruct((_N_PAD, _F), jnp.float32)],
)

_out_call = pl.pallas_call(
    _out_body,
    grid=_GRID,
    in_specs=[_part_spec(0, _F), _part_spec(1, _F), _row_spec(_F), _row_spec(_F),
              _full_spec(_F, _Z), _full_spec(1, _Z), _full_spec(_F, _Z), _full_spec(1, _Z)],
    out_specs=[_row_spec(_Z), _row_spec(_Z)],
    out_shape=[jax.ShapeDtypeStruct((_N, _Z), jnp.float32)] * 2,
)


def kernel(x, adj, W1, b1, Wmu, bmu, Wsig, bsig):
    src = adj[0].astype(jnp.int32)
    dst = adj[1].astype(jnp.int32)
    fill = jnp.full((_E_PAD - _E,), _DUMMY, dtype=jnp.int32)
    src2 = jnp.concatenate([src, fill]).reshape(_NW * _CHUNKS, 128)
    dst2 = jnp.concatenate([dst, fill]).reshape(_NW * _CHUNKS, 128)
    x_p = jnp.pad(x, ((0, _N_PAD - _N), (0, 0)))
    zeros_f = jnp.zeros((_N_PAD, _F), jnp.float32)

    deg2 = _deg_kernel(dst2, zeros_f)
    xs, dv = _scale_call(x_p, deg2, deg2)
    p = _agg_kernel(src2, dst2, xs, zeros_f)
    (hs,) = _hidden_call(p, p, xs, dv, W1, b1.reshape(1, _F))
    q = _agg_kernel(src2, dst2, hs, zeros_f)
    z_loc, z_scale = _out_call(q, q, hs, dv, Wmu, bmu.reshape(1, _Z),
                               Wsig, bsig.reshape(1, _Z))
    return z_loc, z_scale


# speedup vs baseline: 12.6023x; 1.0477x over previous
import functools

import jax
import jax.numpy as jnp
from jax import lax
from jax.experimental import pallas as pl
from jax.experimental.pallas import tpu as pltpu
from jax.experimental.pallas import tpu_sc as plsc

_N = 10000
_E = 320000
_F = 128
_Z = 64
_N_PAD = 10240
_E_PAD = 327680
_NW = 32
_CHUNKS = 80
_ROWS_PER_TILE = _N_PAD // 16
_DUMMY = 10200


_sc_mesh = plsc.VectorSubcoreMesh(core_axis_name="c", subcore_axis_name="s")


@functools.partial(
    pl.kernel,
    out_type=jax.ShapeDtypeStruct((2, _N_PAD, _F), jnp.float32),
    mesh=_sc_mesh,
    scratch_types=[
        pltpu.VMEM((_CHUNKS, 128), jnp.int32),
        pltpu.VMEM((128, _F), jnp.float32),
        pltpu.VMEM_SHARED((_N_PAD, _F), jnp.float32),
    ],
)
def _deg_kernel(dst_hbm, zeros_hbm, out, dst_v, ones_v, acc_sh):
    c = lax.axis_index("c")
    s = lax.axis_index("s")
    w = s * 2 + c
    row0 = s * _ROWS_PER_TILE
    rows = pl.ds(row0, _ROWS_PER_TILE)
    pltpu.sync_copy(zeros_hbm.at[rows], acc_sh.at[rows])

    def ofill(i, carry):
        for m in range(8):
            ones_v[i, pl.ds(m * 16, 16)] = jnp.ones((16,), jnp.float32)
        return carry

    lax.fori_loop(0, 128, ofill, 0)
    pltpu.sync_copy(dst_hbm.at[pl.ds(w * _CHUNKS, _CHUNKS)], dst_v)
    plsc.subcore_barrier()

    def body(j, carry):
        pltpu.sync_copy(ones_v, acc_sh.at[dst_v.at[j]], add=True)
        return carry

    lax.fori_loop(0, _CHUNKS, body, 0)
    plsc.subcore_barrier()
    pltpu.sync_copy(acc_sh.at[rows], out.at[c, rows])


_NBUF = 2
_BLK = 32
_CH0 = 128
_CH1 = 32


@functools.partial(
    pl.kernel,
    out_type=jax.ShapeDtypeStruct((2, _N_PAD, _F), jnp.float32),
    mesh=_sc_mesh,
    scratch_types=[
        pltpu.VMEM((_BLK, 128), jnp.int32),
        pltpu.VMEM((_BLK, 128), jnp.int32),
    ]
    + [pltpu.VMEM((128, _F), jnp.float32) for _ in range(_NBUF)]
    + [pltpu.VMEM_SHARED((_N_PAD, _F), jnp.float32)]
    + [pltpu.SemaphoreType.DMA for _ in range(_NBUF)],
)
def _agg_kernel(src_hbm, dst_hbm, feat_hbm, zeros_hbm, out,
                src_v, dst_v, b0, b1, acc_sh, s0, s1):
    c = lax.axis_index("c")
    s = lax.axis_index("s")
    row0 = s * _ROWS_PER_TILE
    rows = pl.ds(row0, _ROWS_PER_TILE)
    bufs = (b0, b1)
    sems = (s0, s1)
    pltpu.sync_copy(zeros_hbm.at[rows], acc_sh.at[rows])
    plsc.subcore_barrier()

    def do_block(base_row):
        pltpu.sync_copy(src_hbm.at[pl.ds(base_row, _BLK)], src_v)
        pltpu.sync_copy(dst_hbm.at[pl.ds(base_row, _BLK)], dst_v)
        for b in range(_NBUF):
            pltpu.async_copy(feat_hbm.at[src_v.at[b]], bufs[b], sems[b])

        def body(k, carry):
            for b in range(_NBUF):
                j = k * _NBUF + b
                pltpu.make_async_copy(feat_hbm.at[src_v.at[j]], bufs[b],
                                      sems[b]).wait()
                pltpu.sync_copy(bufs[b], acc_sh.at[dst_v.at[j]], add=True)
                jn = j + _NBUF

                @pl.when(jn < _BLK)
                def _():
                    pltpu.async_copy(feat_hbm.at[src_v.at[jn]], bufs[b],
                                     sems[b])

            return carry

        lax.fori_loop(0, _BLK // _NBUF, body, 0)

    @pl.when(c == 0)
    def _():
        for h in range(_CH0 // _BLK):
            do_block(s * _CH0 + h * _BLK)

    @pl.when(c == 1)
    def _():
        do_block(16 * _CH0 + s * _CH1)

    plsc.subcore_barrier()
    pltpu.sync_copy(acc_sh.at[rows], out.at[c, rows])


_BM = 1024
_GRID = (_N_PAD // _BM,)


def _scale_body(x_ref, da_ref, db_ref, xs_ref, dv_ref):
    deg = da_ref[0] + db_ref[0] + 1.0
    dinv = lax.rsqrt(deg)
    xs_ref[...] = x_ref[...] * dinv
    dv_ref[...] = dinv


def _hidden_body(pa_ref, pb_ref, xs_ref, dv_ref, w1_ref, b1_ref, hs_ref):
    dv = dv_ref[...]
    t = (pa_ref[0] + pb_ref[0] + xs_ref[...]) * dv
    h = jnp.dot(t, w1_ref[...], preferred_element_type=jnp.float32) + b1_ref[...]
    hidden = jnp.maximum(h, 0.0) + jnp.log1p(jnp.exp(-jnp.abs(h)))
    hs_ref[...] = hidden * dv


def _out_body(qa_ref, qb_ref, hs_ref, dv_ref, wmu_ref, bmu_ref,
              wsig_ref, bsig_ref, zl_ref, zs_ref):
    t = (qa_ref[0] + qb_ref[0] + hs_ref[...]) * dv_ref[...]
    zl_ref[...] = jnp.dot(t, wmu_ref[...],
                          preferred_element_type=jnp.float32) + bmu_ref[...]
    zs_ref[...] = jnp.exp(jnp.dot(t, wsig_ref[...],
                                  preferred_element_type=jnp.float32) + bsig_ref[...])


def _row_spec(width):
    return pl.BlockSpec((_BM, width), lambda i: (i, 0))


def _part_spec(cidx, width):
    return pl.BlockSpec((1, _BM, width), lambda i, c=cidx: (c, i, 0))


def _full_spec(r, ccols):
    return pl.BlockSpec((r, ccols), lambda i: (0, 0))


_scale_call = pl.pallas_call(
    _scale_body,
    grid=_GRID,
    in_specs=[_row_spec(_F), _part_spec(0, _F), _part_spec(1, _F)],
    out_specs=[_row_spec(_F), _row_spec(_F)],
    out_shape=[jax.ShapeDtypeStruct((_N_PAD, _F), jnp.float32)] * 2,
)

_hidden_call = pl.pallas_call(
    _hidden_body,
    grid=_GRID,
    in_specs=[_part_spec(0, _F), _part_spec(1, _F), _row_spec(_F), _row_spec(_F),
              _full_spec(_F, _F), _full_spec(1, _F)],
    out_specs=[_row_spec(_F)],
    out_shape=[jax.ShapeDtypeStruct((_N_PAD, _F), jnp.float32)],
)

_out_call = pl.pallas_call(
    _out_body,
    grid=_GRID,
    in_specs=[_part_spec(0, _F), _part_spec(1, _F), _row_spec(_F), _row_spec(_F),
              _full_spec(_F, _Z), _full_spec(1, _Z), _full_spec(_F, _Z), _full_spec(1, _Z)],
    out_specs=[_row_spec(_Z), _row_spec(_Z)],
    out_shape=[jax.ShapeDtypeStruct((_N, _Z), jnp.float32)] * 2,
)


def kernel(x, adj, W1, b1, Wmu, bmu, Wsig, bsig):
    src = adj[0].astype(jnp.int32)
    dst = adj[1].astype(jnp.int32)
    fill = jnp.full((_E_PAD - _E,), _DUMMY, dtype=jnp.int32)
    src2 = jnp.concatenate([src, fill]).reshape(_NW * _CHUNKS, 128)
    dst2 = jnp.concatenate([dst, fill]).reshape(_NW * _CHUNKS, 128)
    x_p = jnp.pad(x, ((0, _N_PAD - _N), (0, 0)))
    zeros_f = jnp.zeros((_N_PAD, _F), jnp.float32)

    deg2 = _deg_kernel(dst2, zeros_f)
    xs, dv = _scale_call(x_p, deg2, deg2)
    p = _agg_kernel(src2, dst2, xs, zeros_f)
    (hs,) = _hidden_call(p, p, xs, dv, W1, b1.reshape(1, _F))
    q = _agg_kernel(src2, dst2, hs, zeros_f)
    z_loc, z_scale = _out_call(q, q, hs, dv, Wmu, bmu.reshape(1, _Z),
                               Wsig, bsig.reshape(1, _Z))
    return z_loc, z_scale
